# baseline (speedup 1.0000x reference)
import functools

import jax
import jax.numpy as jnp
from jax import lax
from jax.experimental import pallas as pl
from jax.experimental.pallas import tpu as pltpu
from jax.experimental.pallas import tpu_sc as plsc

N = 10000
FEAT = 128
OUTD = 64
NC, NS = 2, 16
NW = NC * NS
CHUNK = 128
SB = 8
ACC_ROWS = 10112
ZROWS = ACC_ROWS // NS
DEGW = 16

_MESH = dict(core_axis_name="c", subcore_axis_name="s",
             num_cores=NC, num_subcores=NS)


def _deg_body(dst_hbm, ones_hbm, zeros_hbm, out_hbm, dst_v, ones_v, acc):
    cid = lax.axis_index("c")
    sid = lax.axis_index("s")
    w = cid * NS + sid
    nchunks = dst_hbm.shape[1]
    pltpu.sync_copy(dst_hbm.at[w], dst_v)
    pltpu.sync_copy(ones_hbm, ones_v)
    pltpu.sync_copy(zeros_hbm, acc.at[pl.ds(sid * ZROWS, ZROWS)])
    plsc.subcore_barrier()

    def step(j, carry):
        pltpu.sync_copy(ones_v, acc.at[dst_v.at[j]], add=True)
        return carry

    lax.fori_loop(0, nchunks, step, 0)
    plsc.subcore_barrier()
    pltpu.sync_copy(acc.at[pl.ds(sid * ZROWS, ZROWS)],
                    out_hbm.at[cid, pl.ds(sid * ZROWS, ZROWS)])


def _sc_degree(dst3, ones, zeros):
    kern = functools.partial(
        pl.kernel,
        out_type=jax.ShapeDtypeStruct((NC, ACC_ROWS, FEAT), jnp.float32),
        mesh=plsc.VectorSubcoreMesh(**_MESH),
        scratch_types=[
            pltpu.VMEM(dst3.shape[1:], jnp.int32),
            pltpu.VMEM((CHUNK, FEAT), jnp.float32),
            pltpu.VMEM_SHARED((ACC_ROWS, FEAT), jnp.float32),
        ],
    )(_deg_body)
    return kern(dst3, ones, zeros)


def _agg_body(g_hbm, src_hbm, dst_hbm, zeros_hbm,
              out_hbm, sidx_v, didx_v, rows_a, rows_b, acc, sem_a, sem_b):
    cid = lax.axis_index("c")
    sid = lax.axis_index("s")
    bufs = (rows_a, rows_b)
    sems = (sem_a, sem_b)

    @pl.when(cid == 0)
    def _():
        pltpu.sync_copy(zeros_hbm, acc.at[pl.ds(sid * ZROWS, ZROWS)])
        plsc.subcore_barrier()
        nsb = src_hbm.shape[1] // SB

        def sblock(s, carry):
            base = pl.multiple_of(s * SB, SB)
            pltpu.sync_copy(src_hbm.at[sid, pl.ds(base, SB)], sidx_v)
            pltpu.sync_copy(dst_hbm.at[sid, pl.ds(base, SB)], didx_v)
            pltpu.async_copy(g_hbm.at[sidx_v.at[0]], rows_a, sem_a)
            for k in range(SB):
                if k + 1 < SB:
                    pltpu.async_copy(g_hbm.at[sidx_v.at[k + 1]],
                                     bufs[(k + 1) % 2], sems[(k + 1) % 2])
                pltpu.make_async_copy(g_hbm.at[sidx_v.at[k]], bufs[k % 2],
                                      sems[k % 2]).wait()
                pltpu.sync_copy(bufs[k % 2], acc.at[didx_v.at[k]], add=True)
            return carry

        lax.fori_loop(0, nsb, sblock, 0)
        plsc.subcore_barrier()
        pltpu.sync_copy(acc.at[pl.ds(sid * ZROWS, ZROWS)],
                        out_hbm.at[0, pl.ds(sid * ZROWS, ZROWS)])


def _sc_aggregate(g, src3, dst3, zeros):
    assert src3.shape[1] % SB == 0
    kern = functools.partial(
        pl.kernel,
        out_type=jax.ShapeDtypeStruct((1, ACC_ROWS, FEAT), jnp.float32),
        mesh=plsc.VectorSubcoreMesh(**_MESH),
        scratch_types=[
            pltpu.VMEM((SB, CHUNK), jnp.int32),
            pltpu.VMEM((SB, CHUNK), jnp.int32),
            pltpu.VMEM((CHUNK, FEAT), jnp.float32),
            pltpu.VMEM((CHUNK, FEAT), jnp.float32),
            pltpu.VMEM_SHARED((ACC_ROWS, FEAT), jnp.float32),
            pltpu.SemaphoreType.DMA,
            pltpu.SemaphoreType.DMA,
        ],
    )(_agg_body)
    return kern(g, src3, dst3, zeros)


ROWB = 2000


def _dinv_from(dp_ref):
    deg = dp_ref[0, :, 0:1] + dp_ref[1, :, 0:1] + 1.0
    return lax.rsqrt(deg)


def _t1_body(x_ref, we_ref, be_ref, w1_ref, dp_ref, h0_ref, g1_ref):
    h0 = jnp.maximum(
        jnp.dot(x_ref[...], we_ref[...], preferred_element_type=jnp.float32)
        + be_ref[...], 0.0)
    h0_ref[...] = h0
    g1_ref[...] = (_dinv_from(dp_ref) *
                   jnp.dot(h0, w1_ref[...], preferred_element_type=jnp.float32))


def _tc_embed(x, embed_W, embed_b, conv_W1, dp):
    grid = (N // ROWB,)
    return pl.pallas_call(
        _t1_body,
        grid=grid,
        in_specs=[
            pl.BlockSpec((ROWB, FEAT), lambda i: (i, 0)),
            pl.BlockSpec((FEAT, FEAT), lambda i: (0, 0)),
            pl.BlockSpec((1, FEAT), lambda i: (0, 0)),
            pl.BlockSpec((FEAT, FEAT), lambda i: (0, 0)),
            pl.BlockSpec((NC, ROWB, FEAT), lambda i: (0, i, 0)),
        ],
        out_specs=[
            pl.BlockSpec((ROWB, FEAT), lambda i: (i, 0)),
            pl.BlockSpec((ROWB, FEAT), lambda i: (i, 0)),
        ],
        out_shape=[
            jax.ShapeDtypeStruct((N, FEAT), jnp.float32),
            jax.ShapeDtypeStruct((N, FEAT), jnp.float32),
        ],
    )(x, embed_W, embed_b.reshape(1, FEAT), conv_W1, dp)


def _t2_body(p_ref, g_ref, dp_ref, b_ref, w_ref, h_ref, gn_ref):
    dinv = _dinv_from(dp_ref)
    agg = p_ref[0] + g_ref[...]
    h = jnp.maximum(dinv * agg + b_ref[...], 0.0)
    h_ref[...] = h
    gn_ref[...] = dinv * jnp.dot(h, w_ref[...],
                                 preferred_element_type=jnp.float32)


def _tc_mid(p, g, dp, b, W_next):
    grid = (N // ROWB,)
    return pl.pallas_call(
        _t2_body,
        grid=grid,
        in_specs=[
            pl.BlockSpec((1, ROWB, FEAT), lambda i: (0, i, 0)),
            pl.BlockSpec((ROWB, FEAT), lambda i: (i, 0)),
            pl.BlockSpec((NC, ROWB, FEAT), lambda i: (0, i, 0)),
            pl.BlockSpec((1, FEAT), lambda i: (0, 0)),
            pl.BlockSpec((FEAT, FEAT), lambda i: (0, 0)),
        ],
        out_specs=[
            pl.BlockSpec((ROWB, FEAT), lambda i: (i, 0)),
            pl.BlockSpec((ROWB, FEAT), lambda i: (i, 0)),
        ],
        out_shape=[
            jax.ShapeDtypeStruct((N, FEAT), jnp.float32),
            jax.ShapeDtypeStruct((N, FEAT), jnp.float32),
        ],
    )(p, g, dp, b.reshape(1, FEAT), W_next)


def _t3_body(p_ref, g_ref, dp_ref, b_ref, h0_ref, h1_ref, cw_ref, cb_ref,
             out_ref):
    dinv = _dinv_from(dp_ref)
    h2 = jnp.maximum(dinv * (p_ref[0] + g_ref[...]) + b_ref[...],
                     0.0)
    cw = cw_ref[...]
    out = jnp.dot(h0_ref[...], cw[0:FEAT], preferred_element_type=jnp.float32)
    out += jnp.dot(h1_ref[...], cw[FEAT:2 * FEAT],
                   preferred_element_type=jnp.float32)
    out += jnp.dot(h2, cw[2 * FEAT:3 * FEAT],
                   preferred_element_type=jnp.float32)
    out_ref[...] = out + cb_ref[...]


def _tc_final(p, g, dp, b, h0, h1, cls_W, cls_b):
    grid = (N // ROWB,)
    return pl.pallas_call(
        _t3_body,
        grid=grid,
        in_specs=[
            pl.BlockSpec((1, ROWB, FEAT), lambda i: (0, i, 0)),
            pl.BlockSpec((ROWB, FEAT), lambda i: (i, 0)),
            pl.BlockSpec((NC, ROWB, FEAT), lambda i: (0, i, 0)),
            pl.BlockSpec((1, FEAT), lambda i: (0, 0)),
            pl.BlockSpec((ROWB, FEAT), lambda i: (i, 0)),
            pl.BlockSpec((ROWB, FEAT), lambda i: (i, 0)),
            pl.BlockSpec((3 * FEAT, OUTD), lambda i: (0, 0)),
            pl.BlockSpec((1, OUTD), lambda i: (0, 0)),
        ],
        out_specs=pl.BlockSpec((ROWB, OUTD), lambda i: (i, 0)),
        out_shape=jax.ShapeDtypeStruct((N, OUTD), jnp.float32),
    )(p, g, dp, b.reshape(1, FEAT), h0, h1, cls_W, cls_b.reshape(1, OUTD))


def kernel(x, edge_index, embed_W, embed_b, conv_W1, conv_b1,
           conv_W2, conv_b2, cls_W, cls_b):
    src = edge_index[0].astype(jnp.int32)
    dst = edge_index[1].astype(jnp.int32)
    e = src.shape[0]
    ept = -(-e // (NW * CHUNK * SB)) * CHUNK * SB
    pad = NW * ept - e
    src_p = jnp.concatenate([src, jnp.zeros((pad,), jnp.int32)])
    dump = N + (jnp.arange(pad, dtype=jnp.int32) % (ACC_ROWS - N))
    dst_p = jnp.concatenate([dst, dump])
    src3 = src_p.reshape(NW, ept // CHUNK, CHUNK)
    dst3 = dst_p.reshape(NW, ept // CHUNK, CHUNK)

    srcA = src_p.reshape(NS, -1, CHUNK)
    dstA = dst_p.reshape(NS, -1, CHUNK)

    ones128 = jnp.ones((CHUNK, FEAT), jnp.float32)
    zeros128 = jnp.zeros((ZROWS, FEAT), jnp.float32)

    dp = _sc_degree(dst3, ones128, zeros128)
    h0, g1 = _tc_embed(x, embed_W, embed_b, conv_W1, dp)
    p1 = _sc_aggregate(g1, srcA, dstA, zeros128)
    h1, g2 = _tc_mid(p1, g1, dp, conv_b1, conv_W2)
    p2 = _sc_aggregate(g2, srcA, dstA, zeros128)
    out = _tc_final(p2, g2, dp, conv_b2, h0, h1, cls_W, cls_b)
    return out


# speedup vs baseline: 1.4084x; 1.4084x over previous
import functools

import jax
import jax.numpy as jnp
from jax import lax
from jax.experimental import pallas as pl
from jax.experimental.pallas import tpu as pltpu
from jax.experimental.pallas import tpu_sc as plsc

N = 10000
FEAT = 128
OUTD = 64
NC, NS = 2, 16
NW = NC * NS
CHUNK = 128
SB = 8
ACC_ROWS = 10112
ZROWS = ACC_ROWS // NS
DEGW = 16

_MESH = dict(core_axis_name="c", subcore_axis_name="s",
             num_cores=NC, num_subcores=NS)


def _deg_body(dst_hbm, ones_hbm, zeros_hbm, out_hbm, dst_v, ones_v, acc):
    cid = lax.axis_index("c")
    sid = lax.axis_index("s")
    w = cid * NS + sid
    nchunks = dst_hbm.shape[1]
    pltpu.sync_copy(dst_hbm.at[w], dst_v)
    pltpu.sync_copy(ones_hbm, ones_v)
    pltpu.sync_copy(zeros_hbm, acc.at[pl.ds(sid * ZROWS, ZROWS)])
    plsc.subcore_barrier()

    def step(j, carry):
        pltpu.sync_copy(ones_v, acc.at[dst_v.at[j]], add=True)
        return carry

    lax.fori_loop(0, nchunks, step, 0)
    plsc.subcore_barrier()
    pltpu.sync_copy(acc.at[pl.ds(sid * ZROWS, ZROWS)],
                    out_hbm.at[cid, pl.ds(sid * ZROWS, ZROWS)])


def _sc_degree(dst3, ones, zeros):
    kern = functools.partial(
        pl.kernel,
        out_type=jax.ShapeDtypeStruct((NC, ACC_ROWS, FEAT), jnp.float32),
        mesh=plsc.VectorSubcoreMesh(**_MESH),
        scratch_types=[
            pltpu.VMEM(dst3.shape[1:], jnp.int32),
            pltpu.VMEM((CHUNK, FEAT), jnp.float32),
            pltpu.VMEM_SHARED((ACC_ROWS, FEAT), jnp.float32),
        ],
    )(_deg_body)
    return kern(dst3, ones, zeros)


def _agg_body(g_hbm, srcA_hbm, dstA_hbm, srcB_hbm, dstB_hbm, zeros_hbm,
              out_hbm, sidx_v, didx_v, rows_a, rows_b, acc, sem_a, sem_b):
    cid = lax.axis_index("c")
    sid = lax.axis_index("s")
    pltpu.sync_copy(zeros_hbm, acc.at[pl.ds(sid * ZROWS, ZROWS)])
    plsc.subcore_barrier()

    bufs = (rows_a, rows_b)
    sems = (sem_a, sem_b)

    def run_pipelined(src_hbm, dst_hbm):
        nsb = src_hbm.shape[1] // SB

        def sblock(s, carry):
            base = pl.multiple_of(s * SB, SB)
            pltpu.sync_copy(src_hbm.at[sid, pl.ds(base, SB)], sidx_v)
            pltpu.sync_copy(dst_hbm.at[sid, pl.ds(base, SB)], didx_v)
            pltpu.async_copy(g_hbm.at[sidx_v.at[0]], rows_a, sem_a)
            for k in range(SB):
                if k + 1 < SB:
                    pltpu.async_copy(g_hbm.at[sidx_v.at[k + 1]],
                                     bufs[(k + 1) % 2], sems[(k + 1) % 2])
                pltpu.make_async_copy(g_hbm.at[sidx_v.at[k]], bufs[k % 2],
                                      sems[k % 2]).wait()
                pltpu.sync_copy(bufs[k % 2], acc.at[didx_v.at[k]], add=True)
            return carry

        lax.fori_loop(0, nsb, sblock, 0)

    def run_serial(src_hbm, dst_hbm):
        nsb = src_hbm.shape[1] // SB

        def sblock(s, carry):
            base = pl.multiple_of(s * SB, SB)
            pltpu.sync_copy(src_hbm.at[sid, pl.ds(base, SB)], sidx_v)
            pltpu.sync_copy(dst_hbm.at[sid, pl.ds(base, SB)], didx_v)
            for k in range(SB):
                pltpu.async_copy(g_hbm.at[sidx_v.at[k]], rows_a,
                                 sem_a).wait()
                pltpu.sync_copy(rows_a, acc.at[didx_v.at[k]], add=True)
            return carry

        lax.fori_loop(0, nsb, sblock, 0)

    @pl.when(cid == 0)
    def _():
        run_pipelined(srcA_hbm, dstA_hbm)

    @pl.when(cid == 1)
    def _():
        run_serial(srcB_hbm, dstB_hbm)

    plsc.subcore_barrier()
    pltpu.sync_copy(acc.at[pl.ds(sid * ZROWS, ZROWS)],
                    out_hbm.at[cid, pl.ds(sid * ZROWS, ZROWS)])


def _sc_aggregate(g, srcA, dstA, srcB, dstB, zeros):
    assert srcA.shape[1] % SB == 0 and srcB.shape[1] % SB == 0
    kern = functools.partial(
        pl.kernel,
        out_type=jax.ShapeDtypeStruct((NC, ACC_ROWS, FEAT), jnp.float32),
        mesh=plsc.VectorSubcoreMesh(**_MESH),
        scratch_types=[
            pltpu.VMEM((SB, CHUNK), jnp.int32),
            pltpu.VMEM((SB, CHUNK), jnp.int32),
            pltpu.VMEM((CHUNK, FEAT), jnp.float32),
            pltpu.VMEM((CHUNK, FEAT), jnp.float32),
            pltpu.VMEM_SHARED((ACC_ROWS, FEAT), jnp.float32),
            pltpu.SemaphoreType.DMA,
            pltpu.SemaphoreType.DMA,
        ],
    )(_agg_body)
    return kern(g, srcA, dstA, srcB, dstB, zeros)


ROWB = 2000


def _dinv_from(dp_ref):
    deg = dp_ref[0, :, 0:1] + dp_ref[1, :, 0:1] + 1.0
    return lax.rsqrt(deg)


def _t1_body(x_ref, we_ref, be_ref, w1_ref, dp_ref, h0_ref, g1_ref):
    h0 = jnp.maximum(
        jnp.dot(x_ref[...], we_ref[...], preferred_element_type=jnp.float32)
        + be_ref[...], 0.0)
    h0_ref[...] = h0
    g1_ref[...] = (_dinv_from(dp_ref) *
                   jnp.dot(h0, w1_ref[...], preferred_element_type=jnp.float32))


def _tc_embed(x, embed_W, embed_b, conv_W1, dp):
    grid = (N // ROWB,)
    return pl.pallas_call(
        _t1_body,
        grid=grid,
        in_specs=[
            pl.BlockSpec((ROWB, FEAT), lambda i: (i, 0)),
            pl.BlockSpec((FEAT, FEAT), lambda i: (0, 0)),
            pl.BlockSpec((1, FEAT), lambda i: (0, 0)),
            pl.BlockSpec((FEAT, FEAT), lambda i: (0, 0)),
            pl.BlockSpec((NC, ROWB, FEAT), lambda i: (0, i, 0)),
        ],
        out_specs=[
            pl.BlockSpec((ROWB, FEAT), lambda i: (i, 0)),
            pl.BlockSpec((ROWB, FEAT), lambda i: (i, 0)),
        ],
        out_shape=[
            jax.ShapeDtypeStruct((N, FEAT), jnp.float32),
            jax.ShapeDtypeStruct((N, FEAT), jnp.float32),
        ],
    )(x, embed_W, embed_b.reshape(1, FEAT), conv_W1, dp)


def _t2_body(p_ref, g_ref, dp_ref, b_ref, w_ref, h_ref, gn_ref):
    dinv = _dinv_from(dp_ref)
    agg = p_ref[0] + p_ref[1] + g_ref[...]
    h = jnp.maximum(dinv * agg + b_ref[...], 0.0)
    h_ref[...] = h
    gn_ref[...] = dinv * jnp.dot(h, w_ref[...],
                                 preferred_element_type=jnp.float32)


def _tc_mid(p, g, dp, b, W_next):
    grid = (N // ROWB,)
    return pl.pallas_call(
        _t2_body,
        grid=grid,
        in_specs=[
            pl.BlockSpec((NC, ROWB, FEAT), lambda i: (0, i, 0)),
            pl.BlockSpec((ROWB, FEAT), lambda i: (i, 0)),
            pl.BlockSpec((NC, ROWB, FEAT), lambda i: (0, i, 0)),
            pl.BlockSpec((1, FEAT), lambda i: (0, 0)),
            pl.BlockSpec((FEAT, FEAT), lambda i: (0, 0)),
        ],
        out_specs=[
            pl.BlockSpec((ROWB, FEAT), lambda i: (i, 0)),
            pl.BlockSpec((ROWB, FEAT), lambda i: (i, 0)),
        ],
        out_shape=[
            jax.ShapeDtypeStruct((N, FEAT), jnp.float32),
            jax.ShapeDtypeStruct((N, FEAT), jnp.float32),
        ],
    )(p, g, dp, b.reshape(1, FEAT), W_next)


def _t3_body(p_ref, g_ref, dp_ref, b_ref, h0_ref, h1_ref, cw_ref, cb_ref,
             out_ref):
    dinv = _dinv_from(dp_ref)
    h2 = jnp.maximum(dinv * (p_ref[0] + p_ref[1] + g_ref[...]) + b_ref[...],
                     0.0)
    cw = cw_ref[...]
    out = jnp.dot(h0_ref[...], cw[0:FEAT], preferred_element_type=jnp.float32)
    out += jnp.dot(h1_ref[...], cw[FEAT:2 * FEAT],
                   preferred_element_type=jnp.float32)
    out += jnp.dot(h2, cw[2 * FEAT:3 * FEAT],
                   preferred_element_type=jnp.float32)
    out_ref[...] = out + cb_ref[...]


def _tc_final(p, g, dp, b, h0, h1, cls_W, cls_b):
    grid = (N // ROWB,)
    return pl.pallas_call(
        _t3_body,
        grid=grid,
        in_specs=[
            pl.BlockSpec((NC, ROWB, FEAT), lambda i: (0, i, 0)),
            pl.BlockSpec((ROWB, FEAT), lambda i: (i, 0)),
            pl.BlockSpec((NC, ROWB, FEAT), lambda i: (0, i, 0)),
            pl.BlockSpec((1, FEAT), lambda i: (0, 0)),
            pl.BlockSpec((ROWB, FEAT), lambda i: (i, 0)),
            pl.BlockSpec((ROWB, FEAT), lambda i: (i, 0)),
            pl.BlockSpec((3 * FEAT, OUTD), lambda i: (0, 0)),
            pl.BlockSpec((1, OUTD), lambda i: (0, 0)),
        ],
        out_specs=pl.BlockSpec((ROWB, OUTD), lambda i: (i, 0)),
        out_shape=jax.ShapeDtypeStruct((N, OUTD), jnp.float32),
    )(p, g, dp, b.reshape(1, FEAT), h0, h1, cls_W, cls_b.reshape(1, OUTD))


def kernel(x, edge_index, embed_W, embed_b, conv_W1, conv_b1,
           conv_W2, conv_b2, cls_W, cls_b):
    src = edge_index[0].astype(jnp.int32)
    dst = edge_index[1].astype(jnp.int32)
    e = src.shape[0]
    ept = -(-e // (NW * CHUNK * SB)) * CHUNK * SB
    pad = NW * ept - e
    src_p = jnp.concatenate([src, jnp.zeros((pad,), jnp.int32)])
    dump = N + (jnp.arange(pad, dtype=jnp.int32) % (ACC_ROWS - N))
    dst_p = jnp.concatenate([dst, dump])
    src3 = src_p.reshape(NW, ept // CHUNK, CHUNK)
    dst3 = dst_p.reshape(NW, ept // CHUNK, CHUNK)

    e_pad = NW * ept
    grain = NS * CHUNK * SB
    ea = max(grain, min(e_pad - grain, (round(e_pad * 0.75) // grain) * grain))
    srcA = src_p[:ea].reshape(NS, -1, CHUNK)
    dstA = dst_p[:ea].reshape(NS, -1, CHUNK)
    srcB = src_p[ea:].reshape(NS, -1, CHUNK)
    dstB = dst_p[ea:].reshape(NS, -1, CHUNK)

    ones128 = jnp.ones((CHUNK, FEAT), jnp.float32)
    zeros128 = jnp.zeros((ZROWS, FEAT), jnp.float32)

    dp = _sc_degree(dst3, ones128, zeros128)
    h0, g1 = _tc_embed(x, embed_W, embed_b, conv_W1, dp)
    p1 = _sc_aggregate(g1, srcA, dstA, srcB, dstB, zeros128)
    h1, g2 = _tc_mid(p1, g1, dp, conv_b1, conv_W2)
    p2 = _sc_aggregate(g2, srcA, dstA, srcB, dstB, zeros128)
    out = _tc_final(p2, g2, dp, conv_b2, h0, h1, cls_W, cls_b)
    return out


# speedup vs baseline: 1.4622x; 1.0382x over previous
import functools

import jax
import jax.numpy as jnp
from jax import lax
from jax.experimental import pallas as pl
from jax.experimental.pallas import tpu as pltpu
from jax.experimental.pallas import tpu_sc as plsc

N = 10000
FEAT = 128
OUTD = 64
NC, NS = 2, 16
NW = NC * NS
CHUNK = 128
SB = 8
ACC_ROWS = 10112
ZROWS = ACC_ROWS // NS
DEGW = 16

_MESH = dict(core_axis_name="c", subcore_axis_name="s",
             num_cores=NC, num_subcores=NS)


def _deg_body(dst_hbm, ones_hbm, zeros_hbm, out_hbm, dst_v, ones_v, acc):
    cid = lax.axis_index("c")
    sid = lax.axis_index("s")
    w = cid * NS + sid
    nchunks = dst_hbm.shape[1]
    pltpu.sync_copy(dst_hbm.at[w], dst_v)
    pltpu.sync_copy(ones_hbm, ones_v)
    pltpu.sync_copy(zeros_hbm, acc.at[pl.ds(sid * ZROWS, ZROWS)])
    plsc.subcore_barrier()

    def step(j, carry):
        pltpu.sync_copy(ones_v, acc.at[dst_v.at[j]], add=True)
        return carry

    lax.fori_loop(0, nchunks, step, 0)
    plsc.subcore_barrier()
    pltpu.sync_copy(acc.at[pl.ds(sid * ZROWS, ZROWS)],
                    out_hbm.at[cid, pl.ds(sid * ZROWS, ZROWS)])


def _sc_degree(dst3, ones, zeros):
    kern = functools.partial(
        pl.kernel,
        out_type=jax.ShapeDtypeStruct((NC, ACC_ROWS, FEAT), jnp.float32),
        mesh=plsc.VectorSubcoreMesh(**_MESH),
        scratch_types=[
            pltpu.VMEM(dst3.shape[1:], jnp.int32),
            pltpu.VMEM((CHUNK, FEAT), jnp.float32),
            pltpu.VMEM_SHARED((ACC_ROWS, FEAT), jnp.float32),
        ],
    )(_deg_body)
    return kern(dst3, ones, zeros)


def _agg_body(g_hbm, srcA_hbm, dstA_hbm, srcB_hbm, dstB_hbm, zeros_hbm,
              out_hbm, sidx_v, didx_v, sidxb_v, didxb_v, rows_a, rows_b, acc,
              sem_a, sem_b):
    cid = lax.axis_index("c")
    sid = lax.axis_index("s")
    pltpu.sync_copy(zeros_hbm, acc.at[pl.ds(sid * ZROWS, ZROWS)])
    plsc.subcore_barrier()

    bufs = (rows_a, rows_b)
    sems = (sem_a, sem_b)

    def run_pipelined(src_hbm, dst_hbm):
        nsb = src_hbm.shape[1] // SB

        def sblock(s, carry):
            base = pl.multiple_of(s * SB, SB)
            pltpu.sync_copy(src_hbm.at[sid, pl.ds(base, SB)], sidx_v)
            pltpu.sync_copy(dst_hbm.at[sid, pl.ds(base, SB)], didx_v)
            pltpu.async_copy(g_hbm.at[sidx_v.at[0]], rows_a, sem_a)
            for k in range(SB):
                if k + 1 < SB:
                    pltpu.async_copy(g_hbm.at[sidx_v.at[k + 1]],
                                     bufs[(k + 1) % 2], sems[(k + 1) % 2])
                pltpu.make_async_copy(g_hbm.at[sidx_v.at[k]], bufs[k % 2],
                                      sems[k % 2]).wait()
                pltpu.sync_copy(bufs[k % 2], acc.at[didx_v.at[k]], add=True)
            return carry

        lax.fori_loop(0, nsb, sblock, 0)

    def run_serial(src_hbm, dst_hbm):
        pltpu.sync_copy(src_hbm.at[sid], sidxb_v)
        pltpu.sync_copy(dst_hbm.at[sid], didxb_v)

        def step(k, carry):
            pltpu.async_copy(g_hbm.at[sidxb_v.at[k]], rows_a, sem_a).wait()
            pltpu.sync_copy(rows_a, acc.at[didxb_v.at[k]], add=True)
            return carry

        lax.fori_loop(0, src_hbm.shape[1], step, 0)

    @pl.when(cid == 0)
    def _():
        run_pipelined(srcA_hbm, dstA_hbm)

    @pl.when(cid == 1)
    def _():
        run_serial(srcB_hbm, dstB_hbm)

    plsc.subcore_barrier()
    pltpu.sync_copy(acc.at[pl.ds(sid * ZROWS, ZROWS)],
                    out_hbm.at[cid, pl.ds(sid * ZROWS, ZROWS)])


TB = 32


def _sc_aggregate(g, srcA, dstA, srcB, dstB, zeros):
    assert srcA.shape[1] % SB == 0 and srcB.shape[1] == TB
    kern = functools.partial(
        pl.kernel,
        out_type=jax.ShapeDtypeStruct((NC, ACC_ROWS, FEAT), jnp.float32),
        mesh=plsc.VectorSubcoreMesh(**_MESH),
        scratch_types=[
            pltpu.VMEM((SB, CHUNK), jnp.int32),
            pltpu.VMEM((SB, CHUNK), jnp.int32),
            pltpu.VMEM((TB, CHUNK), jnp.int32),
            pltpu.VMEM((TB, CHUNK), jnp.int32),
            pltpu.VMEM((CHUNK, FEAT), jnp.float32),
            pltpu.VMEM((CHUNK, FEAT), jnp.float32),
            pltpu.VMEM_SHARED((ACC_ROWS, FEAT), jnp.float32),
            pltpu.SemaphoreType.DMA,
            pltpu.SemaphoreType.DMA,
        ],
    )(_agg_body)
    return kern(g, srcA, dstA, srcB, dstB, zeros)


ROWB = 2000


def _dinv_from(dp_ref):
    deg = dp_ref[0, :, 0:1] + dp_ref[1, :, 0:1] + 1.0
    return lax.rsqrt(deg)


def _t1_body(x_ref, we_ref, be_ref, w1_ref, dp_ref, h0_ref, g1_ref):
    h0 = jnp.maximum(
        jnp.dot(x_ref[...], we_ref[...], preferred_element_type=jnp.float32)
        + be_ref[...], 0.0)
    h0_ref[...] = h0
    g1_ref[...] = (_dinv_from(dp_ref) *
                   jnp.dot(h0, w1_ref[...], preferred_element_type=jnp.float32))


def _tc_embed(x, embed_W, embed_b, conv_W1, dp):
    grid = (N // ROWB,)
    return pl.pallas_call(
        _t1_body,
        grid=grid,
        in_specs=[
            pl.BlockSpec((ROWB, FEAT), lambda i: (i, 0)),
            pl.BlockSpec((FEAT, FEAT), lambda i: (0, 0)),
            pl.BlockSpec((1, FEAT), lambda i: (0, 0)),
            pl.BlockSpec((FEAT, FEAT), lambda i: (0, 0)),
            pl.BlockSpec((NC, ROWB, FEAT), lambda i: (0, i, 0)),
        ],
        out_specs=[
            pl.BlockSpec((ROWB, FEAT), lambda i: (i, 0)),
            pl.BlockSpec((ROWB, FEAT), lambda i: (i, 0)),
        ],
        out_shape=[
            jax.ShapeDtypeStruct((N, FEAT), jnp.float32),
            jax.ShapeDtypeStruct((N, FEAT), jnp.float32),
        ],
    )(x, embed_W, embed_b.reshape(1, FEAT), conv_W1, dp)


def _t2_body(p_ref, g_ref, dp_ref, b_ref, w_ref, h_ref, gn_ref):
    dinv = _dinv_from(dp_ref)
    agg = p_ref[0] + p_ref[1] + g_ref[...]
    h = jnp.maximum(dinv * agg + b_ref[...], 0.0)
    h_ref[...] = h
    gn_ref[...] = dinv * jnp.dot(h, w_ref[...],
                                 preferred_element_type=jnp.float32)


def _tc_mid(p, g, dp, b, W_next):
    grid = (N // ROWB,)
    return pl.pallas_call(
        _t2_body,
        grid=grid,
        in_specs=[
            pl.BlockSpec((NC, ROWB, FEAT), lambda i: (0, i, 0)),
            pl.BlockSpec((ROWB, FEAT), lambda i: (i, 0)),
            pl.BlockSpec((NC, ROWB, FEAT), lambda i: (0, i, 0)),
            pl.BlockSpec((1, FEAT), lambda i: (0, 0)),
            pl.BlockSpec((FEAT, FEAT), lambda i: (0, 0)),
        ],
        out_specs=[
            pl.BlockSpec((ROWB, FEAT), lambda i: (i, 0)),
            pl.BlockSpec((ROWB, FEAT), lambda i: (i, 0)),
        ],
        out_shape=[
            jax.ShapeDtypeStruct((N, FEAT), jnp.float32),
            jax.ShapeDtypeStruct((N, FEAT), jnp.float32),
        ],
    )(p, g, dp, b.reshape(1, FEAT), W_next)


def _t3_body(p_ref, g_ref, dp_ref, b_ref, h0_ref, h1_ref, cw_ref, cb_ref,
             out_ref):
    dinv = _dinv_from(dp_ref)
    h2 = jnp.maximum(dinv * (p_ref[0] + p_ref[1] + g_ref[...]) + b_ref[...],
                     0.0)
    cw = cw_ref[...]
    out = jnp.dot(h0_ref[...], cw[0:FEAT], preferred_element_type=jnp.float32)
    out += jnp.dot(h1_ref[...], cw[FEAT:2 * FEAT],
                   preferred_element_type=jnp.float32)
    out += jnp.dot(h2, cw[2 * FEAT:3 * FEAT],
                   preferred_element_type=jnp.float32)
    out_ref[...] = out + cb_ref[...]


def _tc_final(p, g, dp, b, h0, h1, cls_W, cls_b):
    grid = (N // ROWB,)
    return pl.pallas_call(
        _t3_body,
        grid=grid,
        in_specs=[
            pl.BlockSpec((NC, ROWB, FEAT), lambda i: (0, i, 0)),
            pl.BlockSpec((ROWB, FEAT), lambda i: (i, 0)),
            pl.BlockSpec((NC, ROWB, FEAT), lambda i: (0, i, 0)),
            pl.BlockSpec((1, FEAT), lambda i: (0, 0)),
            pl.BlockSpec((ROWB, FEAT), lambda i: (i, 0)),
            pl.BlockSpec((ROWB, FEAT), lambda i: (i, 0)),
            pl.BlockSpec((3 * FEAT, OUTD), lambda i: (0, 0)),
            pl.BlockSpec((1, OUTD), lambda i: (0, 0)),
        ],
        out_specs=pl.BlockSpec((ROWB, OUTD), lambda i: (i, 0)),
        out_shape=jax.ShapeDtypeStruct((N, OUTD), jnp.float32),
    )(p, g, dp, b.reshape(1, FEAT), h0, h1, cls_W, cls_b.reshape(1, OUTD))


def kernel(x, edge_index, embed_W, embed_b, conv_W1, conv_b1,
           conv_W2, conv_b2, cls_W, cls_b):
    src = edge_index[0].astype(jnp.int32)
    dst = edge_index[1].astype(jnp.int32)
    e = src.shape[0]
    ept = -(-e // (NW * CHUNK * SB)) * CHUNK * SB
    pad = NW * ept - e
    src_p = jnp.concatenate([src, jnp.zeros((pad,), jnp.int32)])
    dump = N + (jnp.arange(pad, dtype=jnp.int32) % (ACC_ROWS - N))
    dst_p = jnp.concatenate([dst, dump])
    src3 = src_p.reshape(NW, ept // CHUNK, CHUNK)
    dst3 = dst_p.reshape(NW, ept // CHUNK, CHUNK)

    e_pad = NW * ept
    ea = e_pad - NS * CHUNK * TB
    srcA = src_p[:ea].reshape(NS, -1, CHUNK)
    dstA = dst_p[:ea].reshape(NS, -1, CHUNK)
    srcB = src_p[ea:].reshape(NS, -1, CHUNK)
    dstB = dst_p[ea:].reshape(NS, -1, CHUNK)

    ones128 = jnp.ones((CHUNK, FEAT), jnp.float32)
    zeros128 = jnp.zeros((ZROWS, FEAT), jnp.float32)

    dp = _sc_degree(dst3, ones128, zeros128)
    h0, g1 = _tc_embed(x, embed_W, embed_b, conv_W1, dp)
    p1 = _sc_aggregate(g1, srcA, dstA, srcB, dstB, zeros128)
    h1, g2 = _tc_mid(p1, g1, dp, conv_b1, conv_W2)
    p2 = _sc_aggregate(g2, srcA, dstA, srcB, dstB, zeros128)
    out = _tc_final(p2, g2, dp, conv_b2, h0, h1, cls_W, cls_b)
    return out


# speedup vs baseline: 3.5593x; 2.4341x over previous
import functools

import jax
import jax.numpy as jnp
from jax import lax
from jax.experimental import pallas as pl
from jax.experimental.pallas import tpu as pltpu
from jax.experimental.pallas import tpu_sc as plsc

N = 10000
FEAT = 128
OUTD = 64
NC, NS = 2, 16
NW = NC * NS
CHUNK = 128
SB = 8
ACC_ROWS = 10112
ZROWS = ACC_ROWS // NS
DEGW = 16

_MESH = dict(core_axis_name="c", subcore_axis_name="s",
             num_cores=NC, num_subcores=NS)


def _deg_body(dst_hbm, ones_hbm, zeros_hbm, out_hbm, dst_v, ones_v, acc):
    cid = lax.axis_index("c")
    sid = lax.axis_index("s")
    w = cid * NS + sid
    nchunks = dst_hbm.shape[1]
    pltpu.sync_copy(dst_hbm.at[w], dst_v)
    pltpu.sync_copy(ones_hbm, ones_v)
    pltpu.sync_copy(zeros_hbm, acc.at[pl.ds(sid * ZROWS, ZROWS)])
    plsc.subcore_barrier()

    def step(j, carry):
        pltpu.sync_copy(ones_v, acc.at[dst_v.at[j]], add=True)
        return carry

    lax.fori_loop(0, nchunks, step, 0)
    plsc.subcore_barrier()
    pltpu.sync_copy(acc.at[pl.ds(sid * ZROWS, ZROWS)],
                    out_hbm.at[cid, pl.ds(sid * ZROWS, ZROWS)])


def _sc_degree(dst3, ones, zeros):
    kern = functools.partial(
        pl.kernel,
        out_type=jax.ShapeDtypeStruct((NC, ACC_ROWS, FEAT), jnp.float32),
        mesh=plsc.VectorSubcoreMesh(**_MESH),
        scratch_types=[
            pltpu.VMEM(dst3.shape[1:], jnp.int32),
            pltpu.VMEM((CHUNK, FEAT), jnp.float32),
            pltpu.VMEM_SHARED((ACC_ROWS, FEAT), jnp.float32),
        ],
    )(_deg_body)
    return kern(dst3, ones, zeros)


def _agg_body(g_hbm, srcA_hbm, dstA_hbm, srcB_hbm, dstB_hbm, zeros_hbm,
              out_hbm, sidx_v, didx_v, rows_a, rows_b, acc, sem_a, sem_b):
    cid = lax.axis_index("c")
    sid = lax.axis_index("s")
    pltpu.sync_copy(zeros_hbm, acc.at[pl.ds(sid * ZROWS, ZROWS)])
    plsc.subcore_barrier()

    bufs = (rows_a, rows_b)
    sems = (sem_a, sem_b)

    def run_pipelined(src_hbm, dst_hbm):
        nsb = src_hbm.shape[1] // SB

        def sblock(s, carry):
            base = pl.multiple_of(s * SB, SB)
            pltpu.sync_copy(src_hbm.at[sid, pl.ds(base, SB)], sidx_v)
            pltpu.sync_copy(dst_hbm.at[sid, pl.ds(base, SB)], didx_v)
            pltpu.async_copy(g_hbm.at[sidx_v.at[0]], rows_a, sem_a)
            for k in range(SB):
                if k + 1 < SB:
                    pltpu.async_copy(g_hbm.at[sidx_v.at[k + 1]],
                                     bufs[(k + 1) % 2], sems[(k + 1) % 2])
                pltpu.make_async_copy(g_hbm.at[sidx_v.at[k]], bufs[k % 2],
                                      sems[k % 2]).wait()
                pltpu.sync_copy(bufs[k % 2], acc.at[didx_v.at[k]], add=True)
            return carry

        lax.fori_loop(0, nsb, sblock, 0)

    @pl.when(cid == 0)
    def _():
        run_pipelined(srcA_hbm, dstA_hbm)

    @pl.when(cid == 1)
    def _():
        run_pipelined(srcB_hbm, dstB_hbm)

    plsc.subcore_barrier()
    pltpu.sync_copy(acc.at[pl.ds(sid * ZROWS, ZROWS)],
                    out_hbm.at[cid, pl.ds(sid * ZROWS, ZROWS)])


def _sc_aggregate(g, srcA, dstA, srcB, dstB, zeros):
    assert srcA.shape[1] % SB == 0 and srcB.shape[1] % SB == 0
    kern = functools.partial(
        pl.kernel,
        out_type=jax.ShapeDtypeStruct((NC, ACC_ROWS, FEAT), jnp.float32),
        mesh=plsc.VectorSubcoreMesh(**_MESH),
        scratch_types=[
            pltpu.VMEM((SB, CHUNK), jnp.int32),
            pltpu.VMEM((SB, CHUNK), jnp.int32),
            pltpu.VMEM((CHUNK, FEAT), jnp.float32),
            pltpu.VMEM((CHUNK, FEAT), jnp.float32),
            pltpu.VMEM_SHARED((ACC_ROWS, FEAT), jnp.float32),
            pltpu.SemaphoreType.DMA,
            pltpu.SemaphoreType.DMA,
        ],
    )(_agg_body)
    return kern(g, srcA, dstA, srcB, dstB, zeros)


ROWB = 2000


def _dinv_from(dp_ref):
    deg = dp_ref[0, :, 0:1] + dp_ref[1, :, 0:1] + 1.0
    return lax.rsqrt(deg)


def _t1_body(x_ref, we_ref, be_ref, w1_ref, dp_ref, h0_ref, g1_ref):
    h0 = jnp.maximum(
        jnp.dot(x_ref[...], we_ref[...], preferred_element_type=jnp.float32)
        + be_ref[...], 0.0)
    h0_ref[...] = h0
    g1_ref[...] = (_dinv_from(dp_ref) *
                   jnp.dot(h0, w1_ref[...], preferred_element_type=jnp.float32))


def _tc_embed(x, embed_W, embed_b, conv_W1, dp):
    grid = (N // ROWB,)
    return pl.pallas_call(
        _t1_body,
        grid=grid,
        in_specs=[
            pl.BlockSpec((ROWB, FEAT), lambda i: (i, 0)),
            pl.BlockSpec((FEAT, FEAT), lambda i: (0, 0)),
            pl.BlockSpec((1, FEAT), lambda i: (0, 0)),
            pl.BlockSpec((FEAT, FEAT), lambda i: (0, 0)),
            pl.BlockSpec((NC, ROWB, FEAT), lambda i: (0, i, 0)),
        ],
        out_specs=[
            pl.BlockSpec((ROWB, FEAT), lambda i: (i, 0)),
            pl.BlockSpec((ROWB, FEAT), lambda i: (i, 0)),
        ],
        out_shape=[
            jax.ShapeDtypeStruct((N, FEAT), jnp.float32),
            jax.ShapeDtypeStruct((N, FEAT), jnp.float32),
        ],
    )(x, embed_W, embed_b.reshape(1, FEAT), conv_W1, dp)


def _t2_body(p_ref, g_ref, dp_ref, b_ref, w_ref, h_ref, gn_ref):
    dinv = _dinv_from(dp_ref)
    agg = p_ref[0] + p_ref[1] + g_ref[...]
    h = jnp.maximum(dinv * agg + b_ref[...], 0.0)
    h_ref[...] = h
    gn_ref[...] = dinv * jnp.dot(h, w_ref[...],
                                 preferred_element_type=jnp.float32)


def _tc_mid(p, g, dp, b, W_next):
    grid = (N // ROWB,)
    return pl.pallas_call(
        _t2_body,
        grid=grid,
        in_specs=[
            pl.BlockSpec((NC, ROWB, FEAT), lambda i: (0, i, 0)),
            pl.BlockSpec((ROWB, FEAT), lambda i: (i, 0)),
            pl.BlockSpec((NC, ROWB, FEAT), lambda i: (0, i, 0)),
            pl.BlockSpec((1, FEAT), lambda i: (0, 0)),
            pl.BlockSpec((FEAT, FEAT), lambda i: (0, 0)),
        ],
        out_specs=[
            pl.BlockSpec((ROWB, FEAT), lambda i: (i, 0)),
            pl.BlockSpec((ROWB, FEAT), lambda i: (i, 0)),
        ],
        out_shape=[
            jax.ShapeDtypeStruct((N, FEAT), jnp.float32),
            jax.ShapeDtypeStruct((N, FEAT), jnp.float32),
        ],
    )(p, g, dp, b.reshape(1, FEAT), W_next)


def _t3_body(p_ref, g_ref, dp_ref, b_ref, h0_ref, h1_ref, cw_ref, cb_ref,
             out_ref):
    dinv = _dinv_from(dp_ref)
    h2 = jnp.maximum(dinv * (p_ref[0] + p_ref[1] + g_ref[...]) + b_ref[...],
                     0.0)
    cw = cw_ref[...]
    out = jnp.dot(h0_ref[...], cw[0:FEAT], preferred_element_type=jnp.float32)
    out += jnp.dot(h1_ref[...], cw[FEAT:2 * FEAT],
                   preferred_element_type=jnp.float32)
    out += jnp.dot(h2, cw[2 * FEAT:3 * FEAT],
                   preferred_element_type=jnp.float32)
    out_ref[...] = out + cb_ref[...]


def _tc_final(p, g, dp, b, h0, h1, cls_W, cls_b):
    grid = (N // ROWB,)
    return pl.pallas_call(
        _t3_body,
        grid=grid,
        in_specs=[
            pl.BlockSpec((NC, ROWB, FEAT), lambda i: (0, i, 0)),
            pl.BlockSpec((ROWB, FEAT), lambda i: (i, 0)),
            pl.BlockSpec((NC, ROWB, FEAT), lambda i: (0, i, 0)),
            pl.BlockSpec((1, FEAT), lambda i: (0, 0)),
            pl.BlockSpec((ROWB, FEAT), lambda i: (i, 0)),
            pl.BlockSpec((ROWB, FEAT), lambda i: (i, 0)),
            pl.BlockSpec((3 * FEAT, OUTD), lambda i: (0, 0)),
            pl.BlockSpec((1, OUTD), lambda i: (0, 0)),
        ],
        out_specs=pl.BlockSpec((ROWB, OUTD), lambda i: (i, 0)),
        out_shape=jax.ShapeDtypeStruct((N, OUTD), jnp.float32),
    )(p, g, dp, b.reshape(1, FEAT), h0, h1, cls_W, cls_b.reshape(1, OUTD))


def kernel(x, edge_index, embed_W, embed_b, conv_W1, conv_b1,
           conv_W2, conv_b2, cls_W, cls_b):
    src = edge_index[0].astype(jnp.int32)
    dst = edge_index[1].astype(jnp.int32)
    e = src.shape[0]
    ept = -(-e // (NW * CHUNK * SB)) * CHUNK * SB
    pad = NW * ept - e
    pad_src = jnp.arange(pad, dtype=jnp.int32) % N
    src_p = jnp.concatenate([src, pad_src])
    dump = N + (jnp.arange(pad, dtype=jnp.int32) % (ACC_ROWS - N))
    dst_p = jnp.concatenate([dst, dump])
    src3 = src_p.reshape(NW, ept // CHUNK, CHUNK)
    dst3 = dst_p.reshape(NW, ept // CHUNK, CHUNK)

    e_pad = NW * ept
    ea = e_pad // 2
    srcA = src_p[:ea].reshape(NS, -1, CHUNK)
    dstA = dst_p[:ea].reshape(NS, -1, CHUNK)
    srcB = src_p[ea:].reshape(NS, -1, CHUNK)
    dstB = dst_p[ea:].reshape(NS, -1, CHUNK)

    ones128 = jnp.ones((CHUNK, FEAT), jnp.float32)
    zeros128 = jnp.zeros((ZROWS, FEAT), jnp.float32)

    dp = _sc_degree(dst3, ones128, zeros128)
    h0, g1 = _tc_embed(x, embed_W, embed_b, conv_W1, dp)
    p1 = _sc_aggregate(g1, srcA, dstA, srcB, dstB, zeros128)
    h1, g2 = _tc_mid(p1, g1, dp, conv_b1, conv_W2)
    p2 = _sc_aggregate(g2, srcA, dstA, srcB, dstB, zeros128)
    out = _tc_final(p2, g2, dp, conv_b2, h0, h1, cls_W, cls_b)
    return out
